# baseline (speedup 1.0000x reference)
import jax
import jax.numpy as jnp
from jax.experimental import pallas as pl
from jax.experimental.pallas import tpu as pltpu

_G = 8
_K = 4
_NB = 2 * _K
_HSPLIT = 2


def _make_dma_kernel(N, cg, H, W):
    hb = H // _HSPLIT
    nch = N * _G * _HSPLIT

    def dma_kernel(order_ref, x_any, o_any, bufs, in_sems, out_sems):
        def chunk_coords(c):
            n = c // (_G * _HSPLIT)
            r = c % (_G * _HSPLIT)
            return n, r // _HSPLIT, r % _HSPLIT

        def in_copy(c, slot):
            n, i, h = chunk_coords(c)
            src = order_ref[i]
            return pltpu.make_async_copy(
                x_any.at[n, pl.ds(src * cg, cg), pl.ds(h * hb, hb), :],
                bufs.at[slot],
                in_sems.at[slot],
            )

        def out_copy(c, slot):
            n, i, h = chunk_coords(c)
            return pltpu.make_async_copy(
                bufs.at[slot],
                o_any.at[n, pl.ds(i * cg, cg), pl.ds(h * hb, hb), :],
                out_sems.at[slot],
            )

        for c in range(_K):
            in_copy(c, c % _NB).start()

        def body(c, carry):
            slot = c % _NB

            @pl.when(c >= _K)
            def _():
                out_copy(c - _K, (c - _K) % _NB).wait()

            @pl.when(c + _K < nch)
            def _():
                in_copy(c + _K, (c + _K) % _NB).start()

            in_copy(c, slot).wait()
            out_copy(c, slot).start()
            return carry

        jax.lax.fori_loop(0, nch, body, 0)
        for c in range(nch - _K, nch):
            out_copy(c, c % _NB).wait()

    return dma_kernel, hb, nch


def kernel(x):
    N, C, H, W = x.shape
    g = _G
    cg = C // g
    perm = jax.random.permutation(jax.random.key(42), g - 1)
    order = jnp.concatenate(
        [perm, jnp.array([g - 1], dtype=perm.dtype)], axis=0
    ).astype(jnp.int32)
    body, hb, _ = _make_dma_kernel(N, cg, H, W)
    grid_spec = pltpu.PrefetchScalarGridSpec(
        num_scalar_prefetch=1,
        grid=(1,),
        in_specs=[pl.BlockSpec(memory_space=pl.ANY)],
        out_specs=pl.BlockSpec(memory_space=pl.ANY),
        scratch_shapes=[
            pltpu.VMEM((_NB, cg, hb, W), x.dtype),
            pltpu.SemaphoreType.DMA((_NB,)),
            pltpu.SemaphoreType.DMA((_NB,)),
        ],
    )
    return pl.pallas_call(
        body,
        grid_spec=grid_spec,
        out_shape=jax.ShapeDtypeStruct((N, C, H, W), x.dtype),
    )(order, x)
